# baseline (speedup 1.0000x reference)
import functools

import jax
import jax.numpy as jnp
from jax import lax
from jax.experimental import pallas as pl
from jax.experimental.pallas import tpu as pltpu
from jax.experimental.pallas import tpu_sc as plsc

_BATCH = 4096
_DIM = 64
_TJ = 512
_TI = 512
_CH = 16

_SC_TILES = 3
_SC_COLS = _SC_TILES * _TJ

_K1 = 42
_KS2 = 0x1BD11BF0

_ROT_A = (13, 15, 26, 6)
_ROT_B = (17, 29, 16, 24)
_INJ = ((_K1, _KS2 + 1), (_KS2, 2), (0, _K1 + 3), (_K1, _KS2 + 4), (_KS2, 5))


def _uniform_fb(x1init):
    u32 = jnp.uint32

    def rnd(x0, x1, r):
        x0 = x0 + x1
        x1 = (x1 << u32(r)) | (x1 >> u32(32 - r))
        return x0, x1 ^ x0

    x0 = x1init
    x1 = ((x1init << u32(13)) | (x1init >> u32(19))) ^ x1init

    for g in range(5):
        for r in (_ROT_A if g % 2 == 0 else _ROT_B)[(1 if g == 0 else 0):]:
            x0, x1 = rnd(x0, x1, r)
        a, b = _INJ[g]
        if a:
            x0 = x0 + u32(a)
        x1 = x1 + u32(b)
    return ((x0 ^ x1) >> u32(9)) | u32(0x3F800000)


def _neg_gumbel_from_fb(fb):
    f = lax.bitcast_convert_type(fb, jnp.float32) - jnp.float32(1.0)
    return jnp.log(-jnp.log(f))


def _finish_tile(vmin, vidx, idx_ref):
    best = jnp.min(vmin, axis=0, keepdims=True)
    sel = jnp.min(jnp.where(vmin == best, vidx, jnp.int32(1 << 30)), axis=0)
    idx_ref[0, 0, :] = sel


def _sampler_body(x0_ref, x1t_ref, idx_ref, cross_ref):
    jt = pl.program_id(0)
    x1t = x1t_ref[...]
    sq1 = jnp.sum(x1t * x1t, axis=0, keepdims=True)
    col0 = (jt + _SC_TILES) * _TJ
    iot_r = lax.broadcasted_iota(jnp.int32, (_CH, _TJ), 0)
    p_base = ((iot_r << 12)
              + lax.broadcasted_iota(jnp.int32, (_CH, _TJ), 1)
              + (col0 + _K1)).astype(jnp.uint32)

    def row_tile(it, carry):
        r0 = it * _TI
        lhs = x0_ref[pl.ds(r0, _TI), :]
        cross_ref[...] = jnp.dot(lhs, x1t, preferred_element_type=jnp.float32)

        def chunk(c, carry):
            vmin, vidx = carry
            rb = r0 + c * _CH
            x0c = x0_ref[pl.ds(rb, _CH), :]
            sq0 = jnp.sum(x0c * x0c, axis=1, keepdims=True)
            crossc = cross_ref[pl.ds(c * _CH, _CH), :]
            t = sq0 - 2.0 * crossc + sq1
            m = _neg_gumbel_from_fb(_uniform_fb(p_base + jnp.uint32(rb << 12)))
            w = t + m
            upd = w < vmin
            vmin = jnp.where(upd, w, vmin)
            vidx = jnp.where(upd, rb + iot_r, vidx)
            return vmin, vidx

        return lax.fori_loop(0, _TI // _CH, chunk, carry)

    vmin0 = jnp.full((_CH, _TJ), jnp.inf, jnp.float32)
    vidx0 = jnp.zeros((_CH, _TJ), jnp.int32)
    vmin, vidx = lax.fori_loop(0, _BATCH // _TI, row_tile, (vmin0, vidx0))
    _finish_tile(vmin, vidx, idx_ref)


def _sample_idx(x0, x1t, interpret=False):
    n_tiles = _BATCH // _TJ - _SC_TILES
    idx3 = pl.pallas_call(
        _sampler_body,
        grid=(n_tiles,),
        in_specs=[
            pl.BlockSpec((_BATCH, _DIM), lambda j: (0, 0)),
            pl.BlockSpec((_DIM, _TJ), lambda j: (0, j + _SC_TILES)),
        ],
        out_specs=pl.BlockSpec((1, 1, _TJ), lambda j: (j, 0, 0)),
        out_shape=jax.ShapeDtypeStruct((n_tiles, 1, _TJ), jnp.int32),
        scratch_shapes=[pltpu.VMEM((_TI, _TJ), jnp.float32)],
        interpret=interpret,
    )(x0, x1t)
    return idx3.reshape(n_tiles * _TJ)


def _stripe_body(x0_ref, x1t_ref, fb_ref, idx_ref, cross_ref):
    x1t = x1t_ref[...]
    sq1 = jnp.sum(x1t * x1t, axis=0, keepdims=True)
    iot_r = lax.broadcasted_iota(jnp.int32, (_CH, _TJ), 0)

    def row_tile(it, carry):
        r0 = it * _TI
        lhs = x0_ref[pl.ds(r0, _TI), :]
        cross_ref[...] = jnp.dot(lhs, x1t, preferred_element_type=jnp.float32)

        def chunk(c, carry):
            vmin, vidx = carry
            rb = r0 + c * _CH
            x0c = x0_ref[pl.ds(rb, _CH), :]
            sq0 = jnp.sum(x0c * x0c, axis=1, keepdims=True)
            crossc = cross_ref[pl.ds(c * _CH, _CH), :]
            t = sq0 - 2.0 * crossc + sq1
            fbc = fb_ref[pl.ds(rb, _CH), :]
            w = t + _neg_gumbel_from_fb(fbc)
            upd = w < vmin
            vmin = jnp.where(upd, w, vmin)
            vidx = jnp.where(upd, rb + iot_r, vidx)
            return vmin, vidx

        return lax.fori_loop(0, _TI // _CH, chunk, carry)

    vmin0 = jnp.full((_CH, _TJ), jnp.inf, jnp.float32)
    vidx0 = jnp.zeros((_CH, _TJ), jnp.int32)
    vmin, vidx = lax.fori_loop(0, _BATCH // _TI, row_tile, (vmin0, vidx0))
    _finish_tile(vmin, vidx, idx_ref)


def _stripe_idx(x0, x1t, fb, interpret=False):
    idx3 = pl.pallas_call(
        _stripe_body,
        grid=(_SC_TILES,),
        in_specs=[
            pl.BlockSpec((_BATCH, _DIM), lambda j: (0, 0)),
            pl.BlockSpec((_DIM, _TJ), lambda j: (0, j)),
            pl.BlockSpec((_BATCH, _TJ), lambda j: (0, j)),
        ],
        out_specs=pl.BlockSpec((1, 1, _TJ), lambda j: (j, 0, 0)),
        out_shape=jax.ShapeDtypeStruct((_SC_TILES, 1, _TJ), jnp.int32),
        scratch_shapes=[pltpu.VMEM((_TI, _TJ), jnp.float32)],
        interpret=interpret,
    )(x0, x1t, fb)
    return idx3.reshape(_SC_COLS)


def _sc_bits():
    info = plsc.get_sparse_core_info()
    nc, ns = info.num_cores, info.num_subcores
    rows_per_w = _BATCH // (nc * ns)
    mesh = plsc.VectorSubcoreMesh(core_axis_name="c", subcore_axis_name="s")

    @functools.partial(
        pl.kernel, mesh=mesh,
        out_type=jax.ShapeDtypeStruct((_BATCH, _SC_COLS), jnp.int32),
        scratch_types=[
            pltpu.VMEM((8, _SC_COLS), jnp.int32),
        ],
    )
    def k(out_hbm, buf):
        wid = lax.axis_index("s") * nc + lax.axis_index("c")
        row_base = wid * rows_per_w
        iot = lax.iota(jnp.uint32, 16)

        def slab(s, _):
            row0 = row_base + s * 8

            def row_loop(r8, _):
                pbase = ((row0 + r8) * 4096 + _K1).astype(jnp.uint32)

                def cb_loop(cb, _):
                    off = cb * 128
                    for uu in range(8):
                        x1i = iot + (pbase + jnp.uint32(off + uu * 16))
                        fb = _uniform_fb(x1i)
                        buf[r8, pl.ds(off + uu * 16, 16)] = (
                            fb.astype(jnp.int32))
                    return 0

                return lax.fori_loop(0, _SC_COLS // 128, cb_loop, 0)

            lax.fori_loop(0, 8, row_loop, 0)
            pltpu.sync_copy(buf, out_hbm.at[pl.ds(row0, 8), :])
            return 0

        lax.fori_loop(0, rows_per_w // 8, slab, 0)

    return k()


def _sc_gather(table, idx_stripe, idx_main):
    info = plsc.get_sparse_core_info()
    nw = info.num_cores * info.num_subcores
    bpw = _BATCH // nw
    n_stripe_w = _SC_COLS // bpw
    mesh = plsc.VectorSubcoreMesh(core_axis_name="c", subcore_axis_name="s")

    @functools.partial(
        pl.kernel, mesh=mesh,
        out_type=jax.ShapeDtypeStruct((_BATCH, _DIM), jnp.float32),
        compiler_params=pltpu.CompilerParams(use_tc_tiling_on_sc=False),
        scratch_types=[
            pltpu.VMEM((bpw,), jnp.int32),
            pltpu.VMEM((bpw, _DIM), jnp.float32),
            pltpu.SemaphoreType.DMA,
        ],
    )
    def k(table_hbm, idxs_hbm, idxm_hbm, out_hbm, idx_v, rows_v, sem):
        wid = lax.axis_index("s") * info.num_cores + lax.axis_index("c")
        base = wid * bpw

        @pl.when(wid < n_stripe_w)
        def _():
            pltpu.sync_copy(idxs_hbm.at[pl.ds(base, bpw)], idx_v)

        @pl.when(wid >= n_stripe_w)
        def _():
            pltpu.sync_copy(
                idxm_hbm.at[pl.ds(base - _SC_COLS, bpw)], idx_v)

        pltpu.async_copy(table_hbm.at[idx_v], rows_v, sem).wait()
        pltpu.sync_copy(rows_v, out_hbm.at[pl.ds(base, bpw)])

    return k(table, idx_stripe, idx_main)


def kernel(x0, x1):
    x1t = x1.T
    fb = _sc_bits()
    idx_main = _sample_idx(x0, x1t)
    idx_stripe = _stripe_idx(x0, x1t, fb)
    return _sc_gather(x0, idx_stripe, idx_main)


# speedup vs baseline: 1.3616x; 1.3616x over previous
import functools

import jax
import jax.numpy as jnp
from jax import lax
from jax.experimental import pallas as pl
from jax.experimental.pallas import tpu as pltpu
from jax.experimental.pallas import tpu_sc as plsc

_BATCH = 4096
_DIM = 64
_TJ = 512
_TI = 4096
_CH = 32

_SC_TILES = 2
_SC_COLS = _SC_TILES * _TJ

_K1 = 42
_KS2 = 0x1BD11BF0

_ROT_A = (13, 15, 26, 6)
_ROT_B = (17, 29, 16, 24)
_INJ = ((_K1, _KS2 + 1), (_KS2, 2), (0, _K1 + 3), (_K1, _KS2 + 4), (_KS2, 5))


def _uniform_fb(x1init):
    u32 = jnp.uint32

    def rnd(x0, x1, r):
        x0 = x0 + x1
        x1 = (x1 << u32(r)) | (x1 >> u32(32 - r))
        return x0, x1 ^ x0

    x0 = x1init
    x1 = ((x1init << u32(13)) | (x1init >> u32(19))) ^ x1init

    for g in range(5):
        for r in (_ROT_A if g % 2 == 0 else _ROT_B)[(1 if g == 0 else 0):]:
            x0, x1 = rnd(x0, x1, r)
        a, b = _INJ[g]
        if a:
            x0 = x0 + u32(a)
        x1 = x1 + u32(b)
    return ((x0 ^ x1) >> u32(9)) | u32(0x3F800000)


def _neg_gumbel_from_fb(fb):
    f = lax.bitcast_convert_type(fb, jnp.float32) - jnp.float32(1.0)
    return jnp.log(-jnp.log(f))


def _finish_tile(vmin, vidx, idx_ref):
    best = jnp.min(vmin, axis=0, keepdims=True)
    sel = jnp.min(jnp.where(vmin == best, vidx, jnp.int32(1 << 30)), axis=0)
    idx_ref[0, 0, :] = sel


def _sampler_body(x0_ref, x1t_ref, idx_ref, cross_ref):
    jt = pl.program_id(0)
    x1t = x1t_ref[...]
    sq1 = jnp.sum(x1t * x1t, axis=0, keepdims=True)
    col0 = (jt + _SC_TILES) * _TJ
    iot_r = lax.broadcasted_iota(jnp.int32, (_CH, _TJ), 0)
    p_base = ((iot_r << 12)
              + lax.broadcasted_iota(jnp.int32, (_CH, _TJ), 1)
              + (col0 + _K1)).astype(jnp.uint32)

    def row_tile(it, carry):
        r0 = it * _TI
        lhs = x0_ref[pl.ds(r0, _TI), :]
        cross_ref[...] = jnp.dot(lhs, x1t, preferred_element_type=jnp.float32)

        def chunk(c, carry):
            vmin, vidx = carry
            rb = r0 + c * _CH
            x0c = x0_ref[pl.ds(rb, _CH), :]
            sq0 = jnp.sum(x0c * x0c, axis=1, keepdims=True)
            crossc = cross_ref[pl.ds(c * _CH, _CH), :]
            t = sq0 - 2.0 * crossc + sq1
            m = _neg_gumbel_from_fb(_uniform_fb(p_base + jnp.uint32(rb << 12)))
            w = t + m
            upd = w < vmin
            vmin = jnp.where(upd, w, vmin)
            vidx = jnp.where(upd, rb + iot_r, vidx)
            return vmin, vidx

        return lax.fori_loop(0, _TI // _CH, chunk, carry)

    vmin0 = jnp.full((_CH, _TJ), jnp.inf, jnp.float32)
    vidx0 = jnp.zeros((_CH, _TJ), jnp.int32)
    vmin, vidx = lax.fori_loop(0, _BATCH // _TI, row_tile, (vmin0, vidx0))
    _finish_tile(vmin, vidx, idx_ref)


def _sample_idx(x0, x1t, interpret=False):
    n_tiles = _BATCH // _TJ - _SC_TILES
    idx3 = pl.pallas_call(
        _sampler_body,
        grid=(n_tiles,),
        in_specs=[
            pl.BlockSpec((_BATCH, _DIM), lambda j: (0, 0)),
            pl.BlockSpec((_DIM, _TJ), lambda j: (0, j + _SC_TILES)),
        ],
        out_specs=pl.BlockSpec((1, 1, _TJ), lambda j: (j, 0, 0)),
        out_shape=jax.ShapeDtypeStruct((n_tiles, 1, _TJ), jnp.int32),
        scratch_shapes=[pltpu.VMEM((_TI, _TJ), jnp.float32)],
        interpret=interpret,
    )(x0, x1t)
    return idx3.reshape(n_tiles * _TJ)


_CH2 = 128


def _stripe_body(x0_ref, x1t_ref, fb_ref, idx_ref, cross_ref):
    x1t = x1t_ref[...]
    sq1 = jnp.sum(x1t * x1t, axis=0, keepdims=True)
    iot_r = lax.broadcasted_iota(jnp.int32, (_CH2, _TJ), 0)

    def row_tile(it, carry):
        r0 = it * _TI
        lhs = x0_ref[pl.ds(r0, _TI), :]
        cross_ref[...] = jnp.dot(lhs, x1t, preferred_element_type=jnp.float32)

        def chunk(c, carry):
            vmin, vidx = carry
            rb = r0 + c * _CH2
            x0c = x0_ref[pl.ds(rb, _CH2), :]
            sq0 = jnp.sum(x0c * x0c, axis=1, keepdims=True)
            crossc = cross_ref[pl.ds(c * _CH2, _CH2), :]
            t = sq0 - 2.0 * crossc + sq1
            fbc = fb_ref[pl.ds(rb, _CH2), :]
            w = t + _neg_gumbel_from_fb(fbc)
            upd = w < vmin
            vmin = jnp.where(upd, w, vmin)
            vidx = jnp.where(upd, rb + iot_r, vidx)
            return vmin, vidx

        return lax.fori_loop(0, _TI // _CH2, chunk, carry)

    vmin0 = jnp.full((_CH2, _TJ), jnp.inf, jnp.float32)
    vidx0 = jnp.zeros((_CH2, _TJ), jnp.int32)
    vmin, vidx = lax.fori_loop(0, _BATCH // _TI, row_tile, (vmin0, vidx0))
    _finish_tile(vmin, vidx, idx_ref)


def _stripe_idx(x0, x1t, fb, interpret=False):
    idx3 = pl.pallas_call(
        _stripe_body,
        grid=(_SC_TILES,),
        in_specs=[
            pl.BlockSpec((_BATCH, _DIM), lambda j: (0, 0)),
            pl.BlockSpec((_DIM, _TJ), lambda j: (0, j)),
            pl.BlockSpec((_BATCH, _TJ), lambda j: (0, j)),
        ],
        out_specs=pl.BlockSpec((1, 1, _TJ), lambda j: (j, 0, 0)),
        out_shape=jax.ShapeDtypeStruct((_SC_TILES, 1, _TJ), jnp.int32),
        scratch_shapes=[pltpu.VMEM((_TI, _TJ), jnp.float32)],
        interpret=interpret,
    )(x0, x1t, fb)
    return idx3.reshape(_SC_COLS)


def _sc_bits():
    info = plsc.get_sparse_core_info()
    nc, ns = info.num_cores, info.num_subcores
    rows_per_w = _BATCH // (nc * ns)
    mesh = plsc.VectorSubcoreMesh(core_axis_name="c", subcore_axis_name="s")

    @functools.partial(
        pl.kernel, mesh=mesh,
        out_type=jax.ShapeDtypeStruct((_BATCH, _SC_COLS), jnp.int32),
        scratch_types=[
            pltpu.VMEM((8, _SC_COLS), jnp.int32),
        ],
    )
    def k(out_hbm, buf):
        wid = lax.axis_index("s") * nc + lax.axis_index("c")
        row_base = wid * rows_per_w
        iot = lax.iota(jnp.uint32, 16)

        def slab(s, _):
            row0 = row_base + s * 8

            def row_loop(r8, _):
                pbase = ((row0 + r8) * 4096 + _K1).astype(jnp.uint32)

                def cb_loop(cb, _):
                    off = cb * 128
                    for uu in range(8):
                        x1i = iot + (pbase + jnp.uint32(off + uu * 16))
                        fb = _uniform_fb(x1i)
                        buf[r8, pl.ds(off + uu * 16, 16)] = (
                            fb.astype(jnp.int32))
                    return 0

                return lax.fori_loop(0, _SC_COLS // 128, cb_loop, 0)

            lax.fori_loop(0, 8, row_loop, 0)
            pltpu.sync_copy(buf, out_hbm.at[pl.ds(row0, 8), :])
            return 0

        lax.fori_loop(0, rows_per_w // 8, slab, 0)

    return k()


def _sc_gather(table, idx_stripe, idx_main):
    info = plsc.get_sparse_core_info()
    nw = info.num_cores * info.num_subcores
    bpw = _BATCH // nw
    n_stripe_w = _SC_COLS // bpw
    mesh = plsc.VectorSubcoreMesh(core_axis_name="c", subcore_axis_name="s")

    @functools.partial(
        pl.kernel, mesh=mesh,
        out_type=jax.ShapeDtypeStruct((_BATCH, _DIM), jnp.float32),
        compiler_params=pltpu.CompilerParams(use_tc_tiling_on_sc=False),
        scratch_types=[
            pltpu.VMEM((bpw,), jnp.int32),
            pltpu.VMEM((bpw, _DIM), jnp.float32),
            pltpu.SemaphoreType.DMA,
        ],
    )
    def k(table_hbm, idxs_hbm, idxm_hbm, out_hbm, idx_v, rows_v, sem):
        wid = lax.axis_index("s") * info.num_cores + lax.axis_index("c")
        base = wid * bpw

        @pl.when(wid < n_stripe_w)
        def _():
            pltpu.sync_copy(idxs_hbm.at[pl.ds(base, bpw)], idx_v)

        @pl.when(wid >= n_stripe_w)
        def _():
            pltpu.sync_copy(
                idxm_hbm.at[pl.ds(base - _SC_COLS, bpw)], idx_v)

        pltpu.async_copy(table_hbm.at[idx_v], rows_v, sem).wait()
        pltpu.sync_copy(rows_v, out_hbm.at[pl.ds(base, bpw)])

    return k(table, idx_stripe, idx_main)


def kernel(x0, x1):
    x1t = x1.T
    fb = _sc_bits()
    idx_main = _sample_idx(x0, x1t)
    idx_stripe = _stripe_idx(x0, x1t, fb)
    return _sc_gather(x0, idx_stripe, idx_main)
